# Initial kernel scaffold; baseline (speedup 1.0000x reference)
import jax
import jax.numpy as jnp
from jax.experimental import pallas as pl


def kernel(B, rho, t, node_idx, query_idx):
    raise NotImplementedError("write your pallas kernel here")


# speedup vs baseline: 1.9074x; 1.9074x over previous
import jax
import jax.numpy as jnp
from jax import lax
from jax.experimental import pallas as pl
from jax.experimental.pallas import tpu as pltpu
from jax.experimental.pallas import tpu_sc as plsc

D = 256
OMEGA = 0.04
PHI_MAX = 5.0
NC = 2
NS = 16
NW = NC * NS
L = 16
NJ = D // L

N_INJECT = 16384
N_QUERY = 4096
N_INJ_W = N_INJECT // NW
CHUNK = 128
NCHUNK = N_INJ_W // CHUNK
N_Q_W = N_QUERY // NW

_PI = 3.14159265358979323846
_TWO_PI = 2.0 * _PI
_SIN_C = (1.000000000e+00, -1.666666716e-01, 8.333333768e-03, -1.984127011e-04,
          2.755733249e-06, -2.505207242e-08, 1.605426347e-10, -7.583586537e-13,
          2.498001805e-15)
_COS_C = (1.000000000e+00, -5.000000000e-01, 4.166667163e-02, -1.388890552e-03,
          2.480187322e-05, -2.756005131e-07, 2.089865392e-09, -1.161400490e-11,
          5.262457137e-14, -2.220446049e-16)


def _sincos(x):
    xr = lax.rem(x, jnp.float32(_TWO_PI))
    xr = jnp.where(xr < 0, xr + jnp.float32(_TWO_PI), xr)
    y = xr - jnp.float32(_PI)
    u = y * y
    ps = jnp.float32(_SIN_C[-1])
    for c in _SIN_C[-2::-1]:
        ps = ps * u + jnp.float32(c)
    pc = jnp.float32(_COS_C[-1])
    for c in _COS_C[-2::-1]:
        pc = pc * u + jnp.float32(c)
    return -(y * ps), -pc


def _worker_id():
    return lax.axis_index("s") * NC + lax.axis_index("c")


def _inject_body(b_hbm, rho_hbm, t_hbm, nidx_hbm, part_hbm,
                 wr_v, wi_v, idx_v, rows0_v, rows1_v, acc_v, sem0, sem1):
    wid = _worker_id()
    pltpu.sync_copy(rho_hbm.at[wid], wr_v)
    pltpu.sync_copy(t_hbm.at[wid], wi_v)
    pltpu.sync_copy(nidx_hbm.at[wid], idx_v)

    bufs = (rows0_v, rows1_v)
    sems = (sem0, sem1)
    h = pltpu.async_copy(b_hbm.at[idx_v.at[0]], bufs[0], sems[0])

    def wbody(i, carry):
        sl = pl.ds(pl.multiple_of(i * L, L), L)
        amp = wr_v[sl] * jnp.float32(1.0 / 16.0)
        sn, cs = _sincos(jnp.float32(OMEGA) * wi_v[sl])
        wr_v[sl] = amp * cs
        wi_v[sl] = amp * sn
        return carry
    lax.fori_loop(0, N_INJ_W // L, wbody, 0)

    accs = tuple(jnp.zeros((L,), jnp.float32) for _ in range(2 * NJ))
    for k in range(NCHUNK):
        if k + 1 < NCHUNK:
            h_next = pltpu.async_copy(
                b_hbm.at[idx_v.at[k + 1]], bufs[(k + 1) % 2], sems[(k + 1) % 2])
        h.wait()
        cur = bufs[k % 2]

        def gbody(g, acc_t, _k=k, _cur=cur):
            woff = pl.multiple_of(_k * CHUNK + g * L, L)
            wr16 = wr_v[pl.ds(woff, L)]
            wi16 = wi_v[pl.ds(woff, L)]
            out = list(acc_t)
            for e in range(L):
                r = g * L + e
                rows = [_cur[r, pl.ds(j * L, L)] for j in range(NJ)]
                for j in range(NJ):
                    out[j] = out[j] + wr16[e] * rows[j]
                    out[NJ + j] = out[NJ + j] + wi16[e] * rows[j]
            return tuple(out)
        accs = lax.fori_loop(0, CHUNK // L, gbody, accs)
        if k + 1 < NCHUNK:
            h = h_next

    for j in range(2 * NJ):
        acc_v[pl.ds(j * L, L)] = accs[j]
    pltpu.sync_copy(acc_v, part_hbm.at[wid])


def _decode_body(b_hbm, part_hbm, qidx_hbm, out_hbm,
                 part_v, idx_v, rows_v, res_v, sem):
    wid = _worker_id()
    pltpu.sync_copy(qidx_hbm.at[wid], idx_v)
    h = pltpu.async_copy(b_hbm.at[idx_v], rows_v, sem)

    pltpu.sync_copy(part_hbm, part_v)

    def pbody(p, acc_t):
        return tuple(acc_t[j] + part_v[p, pl.ds(j * L, L)] for j in range(2 * NJ))
    accs = lax.fori_loop(
        0, NW, pbody, tuple(jnp.zeros((L,), jnp.float32) for _ in range(2 * NJ)))

    n2v = jnp.zeros((L,), jnp.float32)
    for j in range(2 * NJ):
        n2v = n2v + accs[j] * accs[j]
    n2 = jnp.full((L,), jnp.sum(n2v))
    i = plsc.bitcast(n2, jnp.int32)
    i = jnp.int32(0x5F3759DF) - lax.shift_right_arithmetic(i, 1)
    y = plsc.bitcast(i, jnp.float32)
    half = jnp.float32(0.5) * n2
    for _ in range(3):
        y = y * (jnp.float32(1.5) - half * y * y)
    scale = jnp.where(n2 > jnp.float32(PHI_MAX * PHI_MAX),
                      jnp.float32(PHI_MAX) * y, jnp.float32(1.0))
    rr = [accs[j] * scale for j in range(NJ)]

    h.wait()
    lane0 = lax.iota(jnp.int32, L) == 0

    def qbody(r, carry):
        acc = rr[0] * rows_v[r, pl.ds(0, L)]
        for j in range(1, NJ):
            acc = acc + rr[j] * rows_v[r, pl.ds(j * L, L)]
        val = jnp.full((L,), jnp.sum(acc))
        plsc.store_scatter(res_v, [jnp.full((L,), r, jnp.int32)], val, mask=lane0)
        return carry
    lax.fori_loop(0, N_Q_W, qbody, 0)
    pltpu.sync_copy(res_v, out_hbm.at[wid])


import functools


@functools.lru_cache(maxsize=None)
def _build():
    mesh = plsc.VectorSubcoreMesh(core_axis_name="c", subcore_axis_name="s")
    params = pltpu.CompilerParams(needs_layout_passes=False)
    inject = pl.kernel(
        _inject_body,
        out_type=jax.ShapeDtypeStruct((NW, 2 * D), jnp.float32),
        mesh=mesh,
        compiler_params=params,
        scratch_types=[
            pltpu.VMEM((N_INJ_W,), jnp.float32),
            pltpu.VMEM((N_INJ_W,), jnp.float32),
            pltpu.VMEM((NCHUNK, CHUNK), jnp.int32),
            pltpu.VMEM((CHUNK, D), jnp.float32),
            pltpu.VMEM((CHUNK, D), jnp.float32),
            pltpu.VMEM((2 * D,), jnp.float32),
            pltpu.SemaphoreType.DMA,
            pltpu.SemaphoreType.DMA,
        ],
    )
    decode = pl.kernel(
        _decode_body,
        out_type=jax.ShapeDtypeStruct((NW, N_Q_W), jnp.float32),
        mesh=mesh,
        compiler_params=params,
        scratch_types=[
            pltpu.VMEM((NW, 2 * D), jnp.float32),
            pltpu.VMEM((N_Q_W,), jnp.int32),
            pltpu.VMEM((N_Q_W, D), jnp.float32),
            pltpu.VMEM((N_Q_W,), jnp.float32),
            pltpu.SemaphoreType.DMA,
        ],
    )
    return inject, decode


def kernel(B, rho, t, node_idx, query_idx):
    _inject, _decode = _build()
    rho_r = rho.reshape(NW, N_INJ_W)
    t_r = t.reshape(NW, N_INJ_W)
    nidx = node_idx.reshape(NW, NCHUNK, CHUNK)
    qidx = query_idx.reshape(NW, N_Q_W)
    partials = _inject(B, rho_r, t_r, nidx)
    boosts = _decode(B, partials, qidx)
    return boosts.reshape(N_QUERY)


# speedup vs baseline: 2.0007x; 1.0489x over previous
import jax
import jax.numpy as jnp
from jax import lax
from jax.experimental import pallas as pl
from jax.experimental.pallas import tpu as pltpu
from jax.experimental.pallas import tpu_sc as plsc

D = 256
OMEGA = 0.04
PHI_MAX = 5.0
NC = 2
NS = 16
NW = NC * NS
L = 16
NJ = D // L

N_INJECT = 16384
N_QUERY = 4096
N_INJ_W = N_INJECT // NW
CHUNK = 128
NCHUNK = N_INJ_W // CHUNK
N_Q_W = N_QUERY // NW

_PI = 3.14159265358979323846
_TWO_PI = 2.0 * _PI
_SIN_C = (1.000000000e+00, -1.666666716e-01, 8.333333768e-03, -1.984127011e-04,
          2.755733249e-06, -2.505207242e-08, 1.605426347e-10, -7.583586537e-13,
          2.498001805e-15)
_COS_C = (1.000000000e+00, -5.000000000e-01, 4.166667163e-02, -1.388890552e-03,
          2.480187322e-05, -2.756005131e-07, 2.089865392e-09, -1.161400490e-11,
          5.262457137e-14, -2.220446049e-16)


def _sincos(x):
    xr = lax.rem(x, jnp.float32(_TWO_PI))
    xr = jnp.where(xr < 0, xr + jnp.float32(_TWO_PI), xr)
    y = xr - jnp.float32(_PI)
    u = y * y
    ps = jnp.float32(_SIN_C[-1])
    for c in _SIN_C[-2::-1]:
        ps = ps * u + jnp.float32(c)
    pc = jnp.float32(_COS_C[-1])
    for c in _COS_C[-2::-1]:
        pc = pc * u + jnp.float32(c)
    return -(y * ps), -pc


def _worker_id():
    return lax.axis_index("s") * NC + lax.axis_index("c")


def _inject_body(b_hbm, rho_hbm, t_hbm, nidx_hbm, part_hbm,
                 wr_v, wi_v, idx_v, rows0_v, rows1_v, acc_v, sem0, sem1):
    wid = _worker_id()
    ibase = wid * N_INJ_W
    pltpu.sync_copy(rho_hbm.at[pl.ds(ibase, N_INJ_W)], wr_v)
    pltpu.sync_copy(t_hbm.at[pl.ds(ibase, N_INJ_W)], wi_v)
    pltpu.sync_copy(nidx_hbm.at[pl.ds(ibase, N_INJ_W)], idx_v)

    bufs = (rows0_v, rows1_v)
    sems = (sem0, sem1)
    h = pltpu.async_copy(b_hbm.at[idx_v.at[pl.ds(0, CHUNK)]], bufs[0], sems[0])

    def wbody(i, carry):
        sl = pl.ds(pl.multiple_of(i * L, L), L)
        amp = wr_v[sl] * jnp.float32(1.0 / 16.0)
        sn, cs = _sincos(jnp.float32(OMEGA) * wi_v[sl])
        wr_v[sl] = amp * cs
        wi_v[sl] = amp * sn
        return carry
    lax.fori_loop(0, N_INJ_W // L, wbody, 0)

    accs = tuple(jnp.zeros((L,), jnp.float32) for _ in range(2 * NJ))
    for k in range(NCHUNK):
        if k + 1 < NCHUNK:
            h_next = pltpu.async_copy(
                b_hbm.at[idx_v.at[pl.ds((k + 1) * CHUNK, CHUNK)]],
                bufs[(k + 1) % 2], sems[(k + 1) % 2])
        h.wait()
        cur = bufs[k % 2]

        def rbody(r, acc_t, _k=k, _cur=cur):
            bvec = jnp.full((L,), _k * CHUNK + r, jnp.int32)
            wr = plsc.load_gather(wr_v, [bvec])
            wi = plsc.load_gather(wi_v, [bvec])
            rows = [_cur[r, pl.ds(j * L, L)] for j in range(NJ)]
            out = [acc_t[j] + wr * rows[j] for j in range(NJ)]
            out += [acc_t[NJ + j] + wi * rows[j] for j in range(NJ)]
            return tuple(out)
        accs = lax.fori_loop(0, CHUNK, rbody, accs)
        if k + 1 < NCHUNK:
            h = h_next

    for j in range(2 * NJ):
        acc_v[pl.ds(j * L, L)] = accs[j]
    pltpu.sync_copy(acc_v, part_hbm.at[wid])


def _decode_body(b_hbm, part_hbm, qidx_hbm, out_hbm,
                 part_v, idx_v, rows_v, res_v, sem):
    wid = _worker_id()
    pltpu.sync_copy(qidx_hbm.at[pl.ds(wid * N_Q_W, N_Q_W)], idx_v)
    h = pltpu.async_copy(b_hbm.at[idx_v], rows_v, sem)

    pltpu.sync_copy(part_hbm, part_v)

    def pbody(p, acc_t):
        return tuple(acc_t[j] + part_v[p, pl.ds(j * L, L)] for j in range(2 * NJ))
    accs = lax.fori_loop(
        0, NW, pbody, tuple(jnp.zeros((L,), jnp.float32) for _ in range(2 * NJ)))

    n2v = jnp.zeros((L,), jnp.float32)
    for j in range(2 * NJ):
        n2v = n2v + accs[j] * accs[j]
    n2 = jnp.full((L,), jnp.sum(n2v))
    i = plsc.bitcast(n2, jnp.int32)
    i = jnp.int32(0x5F3759DF) - lax.shift_right_arithmetic(i, 1)
    y = plsc.bitcast(i, jnp.float32)
    half = jnp.float32(0.5) * n2
    for _ in range(3):
        y = y * (jnp.float32(1.5) - half * y * y)
    scale = jnp.where(n2 > jnp.float32(PHI_MAX * PHI_MAX),
                      jnp.float32(PHI_MAX) * y, jnp.float32(1.0))
    rr = [accs[j] * scale for j in range(NJ)]

    h.wait()
    lane0 = lax.iota(jnp.int32, L) == 0
    UNQ = 4

    def qbody(g, carry):
        for e in range(UNQ):
            r = g * UNQ + e
            acc = rr[0] * rows_v[r, pl.ds(0, L)]
            for j in range(1, NJ):
                acc = acc + rr[j] * rows_v[r, pl.ds(j * L, L)]
            val = jnp.full((L,), jnp.sum(acc))
            plsc.store_scatter(
                res_v, [jnp.full((L,), r, jnp.int32)], val, mask=lane0)
        return carry
    lax.fori_loop(0, N_Q_W // UNQ, qbody, 0)
    pltpu.sync_copy(res_v, out_hbm.at[pl.ds(wid * N_Q_W, N_Q_W)])


import functools


@functools.lru_cache(maxsize=None)
def _build():
    mesh = plsc.VectorSubcoreMesh(core_axis_name="c", subcore_axis_name="s")
    params = pltpu.CompilerParams(needs_layout_passes=False)
    inject = pl.kernel(
        _inject_body,
        out_type=jax.ShapeDtypeStruct((NW, 2 * D), jnp.float32),
        mesh=mesh,
        compiler_params=params,
        scratch_types=[
            pltpu.VMEM((N_INJ_W,), jnp.float32),
            pltpu.VMEM((N_INJ_W,), jnp.float32),
            pltpu.VMEM((N_INJ_W,), jnp.int32),
            pltpu.VMEM((CHUNK, D), jnp.float32),
            pltpu.VMEM((CHUNK, D), jnp.float32),
            pltpu.VMEM((2 * D,), jnp.float32),
            pltpu.SemaphoreType.DMA,
            pltpu.SemaphoreType.DMA,
        ],
    )
    decode = pl.kernel(
        _decode_body,
        out_type=jax.ShapeDtypeStruct((N_QUERY,), jnp.float32),
        mesh=mesh,
        compiler_params=params,
        scratch_types=[
            pltpu.VMEM((NW, 2 * D), jnp.float32),
            pltpu.VMEM((N_Q_W,), jnp.int32),
            pltpu.VMEM((N_Q_W, D), jnp.float32),
            pltpu.VMEM((N_Q_W,), jnp.float32),
            pltpu.SemaphoreType.DMA,
        ],
    )
    return inject, decode


def kernel(B, rho, t, node_idx, query_idx):
    _inject, _decode = _build()
    partials = _inject(B, rho, t, node_idx)
    return _decode(B, partials, query_idx)


# speedup vs baseline: 2.1210x; 1.0601x over previous
import functools

import jax
import jax.numpy as jnp
from jax import lax
from jax.experimental import pallas as pl
from jax.experimental.pallas import tpu as pltpu
from jax.experimental.pallas import tpu_sc as plsc

D = 256
OMEGA = 0.04
PHI_MAX = 5.0
NC = 2
NS = 16
NW = NC * NS
L = 16
NJ = D // L

N_INJECT = 16384
N_QUERY = 4096
N_INJ_W = N_INJECT // NW
CHUNK = 128
NCHUNK = N_INJ_W // CHUNK
N_Q_W = N_QUERY // NW

_PI = 3.14159265358979323846
_TWO_PI = 2.0 * _PI
_SIN_C = (1.000000000e+00, -1.666666716e-01, 8.333333768e-03, -1.984127011e-04,
          2.755733249e-06, -2.505207242e-08, 1.605426347e-10, -7.583586537e-13,
          2.498001805e-15)
_COS_C = (1.000000000e+00, -5.000000000e-01, 4.166667163e-02, -1.388890552e-03,
          2.480187322e-05, -2.756005131e-07, 2.089865392e-09, -1.161400490e-11,
          5.262457137e-14, -2.220446049e-16)


def _sincos(x):
    xr = lax.rem(x, jnp.float32(_TWO_PI))
    xr = jnp.where(xr < 0, xr + jnp.float32(_TWO_PI), xr)
    y = xr - jnp.float32(_PI)
    u = y * y
    ps = jnp.float32(_SIN_C[-1])
    for c in _SIN_C[-2::-1]:
        ps = ps * u + jnp.float32(c)
    pc = jnp.float32(_COS_C[-1])
    for c in _COS_C[-2::-1]:
        pc = pc * u + jnp.float32(c)
    return -(y * ps), -pc


def _fused_body(b_hbm, rho_hbm, t_hbm, nidx_hbm, qidx_hbm,
                boosts_hbm, part_hbm,
                wr_v, wi_v, idx_v, qidx_v, rows0_v, rows1_v, qrows_v,
                part_v, res_v, acc_v, sem0, sem1, qsem, bar_sem):
    c = lax.axis_index("c")
    s = lax.axis_index("s")
    wid = s * NC + c
    ibase = wid * N_INJ_W

    pltpu.sync_copy(rho_hbm.at[pl.ds(ibase, N_INJ_W)], wr_v)
    pltpu.sync_copy(t_hbm.at[pl.ds(ibase, N_INJ_W)], wi_v)
    pltpu.sync_copy(nidx_hbm.at[pl.ds(ibase, N_INJ_W)], idx_v)
    pltpu.sync_copy(qidx_hbm.at[pl.ds(wid * N_Q_W, N_Q_W)], qidx_v)

    bufs = (rows0_v, rows1_v)
    sems = (sem0, sem1)
    h = pltpu.async_copy(b_hbm.at[idx_v.at[pl.ds(0, CHUNK)]], bufs[0], sems[0])
    qh = pltpu.async_copy(b_hbm.at[qidx_v], qrows_v, qsem)

    def wbody(i, carry):
        sl = pl.ds(pl.multiple_of(i * L, L), L)
        amp = wr_v[sl] * jnp.float32(1.0 / 16.0)
        sn, cs = _sincos(jnp.float32(OMEGA) * wi_v[sl])
        wr_v[sl] = amp * cs
        wi_v[sl] = amp * sn
        return carry
    lax.fori_loop(0, N_INJ_W // L, wbody, 0)

    accs = tuple(jnp.zeros((L,), jnp.float32) for _ in range(2 * NJ))
    for k in range(NCHUNK):
        if k + 1 < NCHUNK:
            h_next = pltpu.async_copy(
                b_hbm.at[idx_v.at[pl.ds((k + 1) * CHUNK, CHUNK)]],
                bufs[(k + 1) % 2], sems[(k + 1) % 2])
        h.wait()
        cur = bufs[k % 2]

        def rbody(r, acc_t, _k=k, _cur=cur):
            bvec = jnp.full((L,), _k * CHUNK + r, jnp.int32)
            wr = plsc.load_gather(wr_v, [bvec])
            wi = plsc.load_gather(wi_v, [bvec])
            rows = [_cur[r, pl.ds(j * L, L)] for j in range(NJ)]
            out = [acc_t[j] + wr * rows[j] for j in range(NJ)]
            out += [acc_t[NJ + j] + wi * rows[j] for j in range(NJ)]
            return tuple(out)
        accs = lax.fori_loop(0, CHUNK, rbody, accs)
        if k + 1 < NCHUNK:
            h = h_next

    for j in range(2 * NJ):
        acc_v[pl.ds(j * L, L)] = accs[j]
    pltpu.sync_copy(acc_v, part_hbm.at[wid])

    plsc.subcore_barrier()
    pltpu.semaphore_signal(bar_sem, 1, core_index=1 - c)
    pltpu.semaphore_wait(bar_sem, 1)

    pltpu.sync_copy(part_hbm, part_v)

    def pbody(p, acc_t):
        return tuple(acc_t[j] + part_v[p, pl.ds(j * L, L)] for j in range(2 * NJ))
    raccs = lax.fori_loop(
        0, NW, pbody, tuple(jnp.zeros((L,), jnp.float32) for _ in range(2 * NJ)))

    n2v = jnp.zeros((L,), jnp.float32)
    for j in range(2 * NJ):
        n2v = n2v + raccs[j] * raccs[j]
    n2 = jnp.full((L,), jnp.sum(n2v))
    i = plsc.bitcast(n2, jnp.int32)
    i = jnp.int32(0x5F3759DF) - lax.shift_right_arithmetic(i, 1)
    y = plsc.bitcast(i, jnp.float32)
    half = jnp.float32(0.5) * n2
    for _ in range(3):
        y = y * (jnp.float32(1.5) - half * y * y)
    scale = jnp.where(n2 > jnp.float32(PHI_MAX * PHI_MAX),
                      jnp.float32(PHI_MAX) * y, jnp.float32(1.0))
    rr = [raccs[j] * scale for j in range(NJ)]

    qh.wait()
    lane0 = lax.iota(jnp.int32, L) == 0
    UNQ = 4

    def qbody(g, carry):
        for e in range(UNQ):
            r = g * UNQ + e
            acc = rr[0] * qrows_v[r, pl.ds(0, L)]
            for j in range(1, NJ):
                acc = acc + rr[j] * qrows_v[r, pl.ds(j * L, L)]
            val = jnp.full((L,), jnp.sum(acc))
            plsc.store_scatter(
                res_v, [jnp.full((L,), r, jnp.int32)], val, mask=lane0)
        return carry
    lax.fori_loop(0, N_Q_W // UNQ, qbody, 0)
    pltpu.sync_copy(res_v, boosts_hbm.at[pl.ds(wid * N_Q_W, N_Q_W)])


@functools.lru_cache(maxsize=None)
def _build():
    mesh = plsc.VectorSubcoreMesh(core_axis_name="c", subcore_axis_name="s")
    return pl.kernel(
        _fused_body,
        out_type=(
            jax.ShapeDtypeStruct((N_QUERY,), jnp.float32),
            jax.ShapeDtypeStruct((NW, 2 * D), jnp.float32),
        ),
        mesh=mesh,
        compiler_params=pltpu.CompilerParams(needs_layout_passes=False),
        scratch_types=[
            pltpu.VMEM((N_INJ_W,), jnp.float32),
            pltpu.VMEM((N_INJ_W,), jnp.float32),
            pltpu.VMEM((N_INJ_W,), jnp.int32),
            pltpu.VMEM((N_Q_W,), jnp.int32),
            pltpu.VMEM((CHUNK, D), jnp.float32),
            pltpu.VMEM((CHUNK, D), jnp.float32),
            pltpu.VMEM((N_Q_W, D), jnp.float32),
            pltpu.VMEM((NW, 2 * D), jnp.float32),
            pltpu.VMEM((N_Q_W,), jnp.float32),
            pltpu.VMEM((2 * D,), jnp.float32),
            pltpu.SemaphoreType.DMA,
            pltpu.SemaphoreType.DMA,
            pltpu.SemaphoreType.DMA,
            pltpu.SemaphoreType.REGULAR,
        ],
    )


def kernel(B, rho, t, node_idx, query_idx):
    boosts, _ = _build()(B, rho, t, node_idx, query_idx)
    return boosts
